# Initial kernel scaffold; baseline (speedup 1.0000x reference)
import jax
import jax.numpy as jnp
from jax.experimental import pallas as pl


def kernel(emb_nodes, emb_edges, graph_node_index, graph_edge_index, num_graphs, Wn1, bn1, Wn2, bn2, Wn3, bn3, We1, be1, We2, be2, We3, be3):
    raise NotImplementedError("write your pallas kernel here")


# speedup vs baseline: 6.8330x; 6.8330x over previous
import functools

import jax
import jax.numpy as jnp
import numpy as np
from jax import lax
from jax.experimental import pallas as pl
from jax.experimental.pallas import tpu as pltpu
from jax.experimental.pallas import tpu_sc as plsc

NUM_SEG = 256
DIM = 128
LANES = 16
VPR = DIM // LANES
NC = 2
NS = 16
NTILES = NC * NS
TBL = NUM_SEG + LANES
NEG = np.float32(-1e30)


def _mlp_logits(x, W1, b1, W2, b2, W3, b3, block=512):
    n = x.shape[0]

    def body(x_ref, w1_ref, b1_ref, w2_ref, b2_ref, w3_ref, b3_ref, out_ref):
        h = jnp.maximum(
            jnp.dot(x_ref[...], w1_ref[...], preferred_element_type=jnp.float32)
            + b1_ref[...], 0.0)
        h = jnp.maximum(
            jnp.dot(h, w2_ref[...], preferred_element_type=jnp.float32)
            + b2_ref[...], 0.0)
        out_ref[...] = (
            jnp.dot(h, w3_ref[...], preferred_element_type=jnp.float32)
            + b3_ref[...])

    full = lambda s: pl.BlockSpec(s, lambda i: (0, 0))
    return pl.pallas_call(
        body,
        grid=(n // block,),
        in_specs=[
            pl.BlockSpec((block, DIM), lambda i: (i, 0)),
            full(W1.shape), full((1, W1.shape[1])),
            full(W2.shape), full((1, W2.shape[1])),
            full(W3.shape), full((1, W3.shape[1])),
        ],
        out_specs=pl.BlockSpec((block, 1), lambda i: (i, 0)),
        out_shape=jax.ShapeDtypeStruct((n, 1), jnp.float32),
    )(x, W1, b1.reshape(1, -1), W2, b2.reshape(1, -1), W3, b3.reshape(1, -1))


def _sc_segment_pool(x, logits, seg, chunk):
    n = x.shape[0]
    rpt = n // NTILES
    nchunks = rpt // chunk
    mesh = plsc.VectorSubcoreMesh(
        core_axis_name="c", subcore_axis_name="s",
        num_cores=NC, num_subcores=NS)

    out_types = (
        jax.ShapeDtypeStruct((NTILES, TBL * DIM), jnp.float32),
        jax.ShapeDtypeStruct((NTILES, TBL * LANES), jnp.float32),
        jax.ShapeDtypeStruct((NTILES, TBL * LANES), jnp.float32),
    )
    scratch = [
        pltpu.VMEM((chunk, DIM), jnp.float32),
        pltpu.VMEM((chunk,), jnp.float32),
        pltpu.VMEM((chunk,), jnp.int32),
        pltpu.VMEM((TBL * DIM,), jnp.float32),
        pltpu.VMEM((TBL * LANES,), jnp.float32),
        pltpu.VMEM((TBL * LANES,), jnp.float32),
    ]

    @functools.partial(pl.kernel, out_type=out_types, mesh=mesh,
                       scratch_types=scratch)
    def k(x_hbm, l_hbm, s_hbm, acc_hbm, m_hbm, d_hbm,
          xb, lb, sb, acc_t, m_t, d_t):
        wid = lax.axis_index("s") * NC + lax.axis_index("c")
        base = wid * rpt
        zeros16 = jnp.zeros((LANES,), jnp.float32)

        def zrow(r, c):
            acc_t[pl.ds(r * LANES, LANES)] = zeros16
            return c
        lax.fori_loop(0, TBL * VPR, zrow, 0, unroll=False)

        def zvec(r, c):
            m_t[pl.ds(r * LANES, LANES)] = jnp.full((LANES,), NEG, jnp.float32)
            d_t[pl.ds(r * LANES, LANES)] = zeros16
            return c
        lax.fori_loop(0, TBL, zvec, 0, unroll=False)

        def flush(prev, m, d, a):
            off = prev * DIM
            for j in range(VPR):
                acc_t[pl.ds(off + j * LANES, LANES)] = a[j]
            m_t[pl.ds(prev * LANES, LANES)] = jnp.full((LANES,), m,
                                                       jnp.float32)
            d_t[pl.ds(prev * LANES, LANES)] = d

        def chunk_body(kc, carry):
            off = base + kc * chunk
            pltpu.sync_copy(x_hbm.at[pl.ds(off, chunk)], xb)
            pltpu.sync_copy(l_hbm.at[pl.ds(off, chunk)], lb)
            pltpu.sync_copy(s_hbm.at[pl.ds(off, chunk)], sb)

            def row_update(i, s, l, c):
                prev, m, d, a = c
                is_new = s != prev

                @pl.when(is_new)
                def _():
                    flush(prev, m, d, a)

                m = jnp.where(is_new, NEG, m)
                d = jnp.where(is_new, 0.0, d)
                a = tuple(jnp.where(is_new, 0.0, aj) for aj in a)
                m_new = jnp.maximum(m, l)
                scale = jnp.exp(jnp.full((LANES,), m - m_new, jnp.float32))
                w = jnp.exp(jnp.full((LANES,), l - m_new, jnp.float32))
                a = tuple(aj * scale + w * xb[i, pl.ds(j * LANES, LANES)]
                          for j, aj in enumerate(a))
                d = d * scale + w
                return (s, m_new, d, a)

            def group_body(g, c):
                sv = sb[pl.ds(g * LANES, LANES)]
                lv = lb[pl.ds(g * LANES, LANES)]
                for j in range(LANES):
                    c = row_update(g * LANES + j, sv[j], lv[j], c)
                return c

            return lax.fori_loop(0, chunk // LANES, group_body, carry,
                                 unroll=False)

        init = (np.int32(NUM_SEG), NEG, zeros16,
                tuple(zeros16 for _ in range(VPR)))
        prev, m, d, a = lax.fori_loop(0, nchunks, chunk_body, init,
                                      unroll=False)
        flush(prev, m, d, a)

        pltpu.sync_copy(acc_t, acc_hbm.at[wid])
        pltpu.sync_copy(m_t, m_hbm.at[wid])
        pltpu.sync_copy(d_t, d_hbm.at[wid])

    return k(x, logits, seg)


def _merge(acc, m, d):
    def body(acc_ref, m_ref, d_ref, out_ref):
        m_all = m_ref[:, :NUM_SEG]
        M = jnp.max(m_all, axis=0, keepdims=True)
        scale = jnp.exp(m_all - M)
        den = jnp.sum(scale * d_ref[:, :NUM_SEG], axis=0)
        num = jnp.sum(scale[:, :, None] * acc_ref[:, :NUM_SEG, :], axis=0)
        den = den[:, None]
        out_ref[...] = jnp.where(den > 0, num / jnp.where(den > 0, den, 1.0),
                                 0.0)

    return pl.pallas_call(
        body,
        out_shape=jax.ShapeDtypeStruct((NUM_SEG, DIM), jnp.float32),
    )(acc, m, d)


def _pool_one(x, seg, W1, b1, W2, b2, W3, b3, chunk):
    logits = _mlp_logits(x, W1, b1, W2, b2, W3, b3)
    acc, m, d = _sc_segment_pool(x, logits.reshape(-1), seg, chunk)
    return _merge(acc.reshape(NTILES, TBL, DIM),
                  m.reshape(NTILES, TBL, LANES)[:, :, 0],
                  d.reshape(NTILES, TBL, LANES)[:, :, 0])


def kernel(emb_nodes, emb_edges, graph_node_index, graph_edge_index,
           num_graphs, Wn1, bn1, Wn2, bn2, Wn3, bn3,
           We1, be1, We2, be2, We3, be3):
    n_nodes = emb_nodes.shape[0]
    n_edges = emb_edges.shape[0]

    node_chunk = 80
    quantum = NTILES * node_chunk * 2
    n_pad = -(-n_nodes // quantum) * quantum
    xn = jnp.pad(emb_nodes, ((0, n_pad - n_nodes), (0, 0)))
    sn = jnp.pad(graph_node_index.astype(jnp.int32), (0, n_pad - n_nodes),
                 constant_values=NUM_SEG)

    edge_chunk = 400
    assert n_edges % (NTILES * edge_chunk) == 0

    out_n = _pool_one(xn, sn, Wn1, bn1, Wn2, bn2, Wn3, bn3, node_chunk)
    out_e = _pool_one(emb_edges, graph_edge_index.astype(jnp.int32),
                      We1, be1, We2, be2, We3, be3, edge_chunk)
    return (out_n, out_e)
